# Initial kernel scaffold; baseline (speedup 1.0000x reference)
import jax
import jax.numpy as jnp
from jax.experimental import pallas as pl


def kernel(z, pos, batch, emb, i_mlp_w1, i_mlp_b1, i_mlp_w2, i_mlp_b2, i_lin1_w, i_lin2_w, i_lin2_b, i_lin_w, i_lin_b, out1_w, out1_b, out2_w, out2_b):
    raise NotImplementedError("write your pallas kernel here")


# speedup vs baseline: 5.2712x; 5.2712x over previous
import functools
import math

import jax
import jax.numpy as jnp
from jax import lax
from jax.experimental import pallas as pl
from jax.experimental.pallas import tpu as pltpu
from jax.experimental.pallas import tpu_sc as plsc

R = 256
CC = 256
MAXNB = 32
CUTOFF = 10.0
GPAD = 128
ZPAD = 128


def _ssp(x):
    return jax.nn.softplus(x) - math.log(2.0)


def _neighbors_body(bounds_ref, pos_r_ref, pos_c_ref, batch_r_ref,
                    idx_ref, d_ref, keep_ref):
    b = pl.program_id(0)
    c0 = bounds_ref[b, 0]
    nch = bounds_ref[b, 1]

    pr = pos_r_ref[...]
    xr = pr[:, 0:1]
    yr = pr[:, 1:2]
    zr = pr[:, 2:3]
    sqr = xr * xr + yr * yr + zr * zr
    br = batch_r_ref[...]
    row_gid = b * R + lax.broadcasted_iota(jnp.int32, (R, 1), 0)

    big_pos = jnp.int32(2**30)
    lane32 = lax.broadcasted_iota(jnp.int32, (R, MAXNB), 1)
    posio = lax.broadcasted_iota(jnp.int32, (R, MAXNB + CC), 1)

    def chunk_body(j, carry):
        bd, bi = carry
        cs = pl.multiple_of(c0 + j * CC, CC)
        pc = pos_c_ref[:, pl.ds(cs, CC)]
        xc = pc[0:1, :]
        yc = pc[1:2, :]
        zc = pc[2:3, :]
        bc = pc[3:4, :]
        sqc = xc * xc + yc * yc + zc * zc
        d2 = sqr + sqc - 2.0 * (xr * xc + yr * yc + zr * zc)
        col_gid = cs + lax.broadcasted_iota(jnp.int32, (1, CC), 1)
        valid = ((br == bc) & (row_gid != col_gid)
                 & (d2 < CUTOFF * CUTOFF))
        cand_d2 = jnp.where(valid, d2, jnp.inf)
        cand_idx = jnp.broadcast_to(col_gid, (R, CC))

        v_d2 = jnp.concatenate([bd, cand_d2], axis=1)
        v_idx = jnp.concatenate([bi, cand_idx], axis=1)
        nb_d2 = jnp.full((R, MAXNB), jnp.inf, jnp.float32)
        nb_idx = jnp.zeros((R, MAXNB), jnp.int32)
        for i in range(MAXNB):
            m = jnp.min(v_d2, axis=1, keepdims=True)
            p = jnp.min(jnp.where(v_d2 == m, posio, big_pos),
                        axis=1, keepdims=True)
            chosen = posio == p
            ic = jnp.min(jnp.where(chosen, v_idx, big_pos),
                         axis=1, keepdims=True)
            nb_d2 = jnp.where(lane32 == i, m, nb_d2)
            nb_idx = jnp.where(lane32 == i, ic, nb_idx)
            v_d2 = jnp.where(chosen, jnp.inf, v_d2)
        return nb_d2, nb_idx

    init = (jnp.full((R, MAXNB), jnp.inf, jnp.float32),
            jnp.zeros((R, MAXNB), jnp.int32))
    bd, bi = lax.fori_loop(0, nch, chunk_body, init)

    keep = bd < jnp.inf
    idx_ref[...] = bi
    keep_ref[...] = keep.astype(jnp.float32)
    d = jnp.sqrt(jnp.maximum(bd, 1e-12))
    d_ref[...] = jnp.where(keep, d, 0.0)


def _neighbors(pos_pad, batch_pad_f, bounds, np_, nb):
    grid_spec = pltpu.PrefetchScalarGridSpec(
        num_scalar_prefetch=1,
        grid=(nb,),
        in_specs=[
            pl.BlockSpec((R, 3), lambda b, s: (b, 0)),
            pl.BlockSpec((8, np_), lambda b, s: (0, 0)),
            pl.BlockSpec((R, 1), lambda b, s: (b, 0)),
        ],
        out_specs=[
            pl.BlockSpec((R, MAXNB), lambda b, s: (b, 0)),
            pl.BlockSpec((R, MAXNB), lambda b, s: (b, 0)),
            pl.BlockSpec((R, MAXNB), lambda b, s: (b, 0)),
        ],
    )
    pos_cols = jnp.concatenate(
        [pos_pad.T, batch_pad_f.T, jnp.zeros((4, np_), jnp.float32)], axis=0)
    return pl.pallas_call(
        _neighbors_body,
        grid_spec=grid_spec,
        out_shape=[
            jax.ShapeDtypeStruct((np_, MAXNB), jnp.int32),
            jax.ShapeDtypeStruct((np_, MAXNB), jnp.float32),
            jax.ShapeDtypeStruct((np_, MAXNB), jnp.float32),
        ],
    )(bounds, pos_pad, pos_cols, batch_pad_f)


def _embed_body(z_ref, emb_ref, lin1_ref, h_ref, x1_ref):
    zc = z_ref[...]
    zio = lax.broadcasted_iota(jnp.int32, (1, ZPAD), 1).astype(jnp.float32)
    onehot = (zc == zio).astype(jnp.float32)
    h = jnp.dot(onehot, emb_ref[...],
                preferred_element_type=jnp.float32)
    h_ref[...] = h
    x1_ref[...] = jnp.dot(h, lin1_ref[...],
                          preferred_element_type=jnp.float32)


def _embed(z_pad_f, emb_pad, lin1_0, np_, nb, hdim):
    return pl.pallas_call(
        _embed_body,
        grid=(nb,),
        in_specs=[
            pl.BlockSpec((R, 1), lambda b: (b, 0)),
            pl.BlockSpec((ZPAD, hdim), lambda b: (0, 0)),
            pl.BlockSpec((hdim, hdim), lambda b: (0, 0)),
        ],
        out_specs=[
            pl.BlockSpec((R, hdim), lambda b: (b, 0)),
            pl.BlockSpec((R, hdim), lambda b: (b, 0)),
        ],
        out_shape=[
            jax.ShapeDtypeStruct((np_, hdim), jnp.float32),
            jax.ShapeDtypeStruct((np_, hdim), jnp.float32),
        ],
    )(z_pad_f, emb_pad, lin1_0)


def _sc_gather(table, idx, hdim):
    btot = idx.shape[0]
    info = plsc.get_sparse_core_info()
    nw = info.num_cores * info.num_subcores
    b_per_w = btot // nw
    ch = 512
    n_chunks = b_per_w // ch
    mesh = plsc.VectorSubcoreMesh(core_axis_name="c", subcore_axis_name="s")

    @functools.partial(
        pl.kernel,
        out_type=jax.ShapeDtypeStruct((btot, hdim), jnp.float32),
        mesh=mesh,
        scratch_types=[
            pltpu.VMEM((ch,), jnp.int32),
            pltpu.VMEM((ch, hdim), jnp.float32),
            pltpu.SemaphoreType.DMA,
        ],
        compiler_params=pltpu.CompilerParams(use_tc_tiling_on_sc=False),
    )
    def gather_kernel(table_hbm, idx_hbm, out_hbm, idx_v, rows_v, sem):
        wid = lax.axis_index("s") * info.num_cores + lax.axis_index("c")
        base = wid * b_per_w

        def body(c, carry):
            off = base + c * ch
            pltpu.sync_copy(idx_hbm.at[pl.ds(off, ch)], idx_v)
            pltpu.async_copy(table_hbm.at[idx_v], rows_v, sem).wait()
            pltpu.sync_copy(rows_v, out_hbm.at[pl.ds(off, ch)])
            return carry

        lax.fori_loop(0, n_chunks, body, 0)

    return gather_kernel(table, idx)


def _interaction_body(g_ref, d_ref, keep_ref, h_ref, off_ref, w1_ref, b1_ref,
                      w2_ref, b2_ref, lin2w_ref, lin2b_ref, linw_ref,
                      linb_ref, lin1n_ref, h_out_ref, x1_out_ref, *,
                      ngauss, hdim, want_x1):
    dcol = d_ref[...]
    off = off_ref[...]
    step = CUTOFF / (ngauss - 1)
    coeff = -0.5 / (step * step)
    ea = jnp.exp(coeff * (dcol - off) ** 2)
    t1 = _ssp(jnp.dot(ea, w1_ref[...],
                      preferred_element_type=jnp.float32) + b1_ref[...])
    w = jnp.dot(t1, w2_ref[...],
                preferred_element_type=jnp.float32) + b2_ref[...]
    c = 0.5 * (jnp.cos(dcol * (math.pi / CUTOFF)) + 1.0)
    w = w * c
    w = jnp.where(keep_ref[...] > 0.0, w, 0.0)
    msg = g_ref[...] * w
    aggr = jnp.sum(msg.reshape(R, MAXNB, hdim), axis=1)
    conv = jnp.dot(aggr, lin2w_ref[...],
                   preferred_element_type=jnp.float32) + lin2b_ref[...]
    hn = h_ref[...] + jnp.dot(_ssp(conv), linw_ref[...],
                              preferred_element_type=jnp.float32) + linb_ref[...]
    h_out_ref[...] = hn
    if want_x1:
        x1_out_ref[...] = jnp.dot(hn, lin1n_ref[...],
                                  preferred_element_type=jnp.float32)


def _interaction(g, d_flat, keep_flat, h, offsets, w1, b1, w2, b2,
                 lin2w, lin2b, linw, linb, lin1n, np_, nb, ngauss, hdim,
                 want_x1):
    er = R * MAXNB
    out_shape = [jax.ShapeDtypeStruct((np_, hdim), jnp.float32)]
    out_specs = [pl.BlockSpec((R, hdim), lambda b: (b, 0))]
    if want_x1:
        out_shape.append(jax.ShapeDtypeStruct((np_, hdim), jnp.float32))
        out_specs.append(pl.BlockSpec((R, hdim), lambda b: (b, 0)))
    body = functools.partial(_interaction_body, ngauss=ngauss, hdim=hdim,
                             want_x1=want_x1)
    if not want_x1:
        def body2(g_ref, d_ref, keep_ref, h_ref, off_ref, w1_ref, b1_ref,
                  w2_ref, b2_ref, lin2w_ref, lin2b_ref, linw_ref, linb_ref,
                  lin1n_ref, h_out_ref):
            body(g_ref, d_ref, keep_ref, h_ref, off_ref, w1_ref, b1_ref,
                 w2_ref, b2_ref, lin2w_ref, lin2b_ref, linw_ref, linb_ref,
                 lin1n_ref, h_out_ref, None)
        kfn = body2
    else:
        kfn = body
    outs = pl.pallas_call(
        kfn,
        grid=(nb,),
        in_specs=[
            pl.BlockSpec((er, hdim), lambda b: (b, 0)),
            pl.BlockSpec((er, 1), lambda b: (b, 0)),
            pl.BlockSpec((er, 1), lambda b: (b, 0)),
            pl.BlockSpec((R, hdim), lambda b: (b, 0)),
            pl.BlockSpec((1, ngauss), lambda b: (0, 0)),
            pl.BlockSpec((ngauss, hdim), lambda b: (0, 0)),
            pl.BlockSpec((1, hdim), lambda b: (0, 0)),
            pl.BlockSpec((hdim, hdim), lambda b: (0, 0)),
            pl.BlockSpec((1, hdim), lambda b: (0, 0)),
            pl.BlockSpec((hdim, hdim), lambda b: (0, 0)),
            pl.BlockSpec((1, hdim), lambda b: (0, 0)),
            pl.BlockSpec((hdim, hdim), lambda b: (0, 0)),
            pl.BlockSpec((1, hdim), lambda b: (0, 0)),
            pl.BlockSpec((hdim, hdim), lambda b: (0, 0)),
        ],
        out_specs=out_specs,
        out_shape=out_shape,
    )(g, d_flat, keep_flat, h, offsets, w1, b1, w2, b2, lin2w, lin2b,
      linw, linb, lin1n)
    if want_x1:
        return outs
    return outs[0], None


def _readout_body(h_ref, batch_ref, w1_ref, b1_ref, w2_ref, b2_ref, out_ref):
    @pl.when(pl.program_id(0) == 0)
    def _():
        out_ref[...] = jnp.zeros_like(out_ref)

    s = _ssp(jnp.dot(h_ref[...], w1_ref[...],
                     preferred_element_type=jnp.float32) + b1_ref[...])
    s = jnp.dot(s, w2_ref[...],
                preferred_element_type=jnp.float32) + b2_ref[...]
    gio = lax.broadcasted_iota(jnp.int32, (1, GPAD), 1).astype(jnp.float32)
    onehot = (batch_ref[...] == gio).astype(jnp.float32)
    out_ref[...] += jnp.sum(onehot * s, axis=0, keepdims=True)


def _readout(h, batch_pad_f, out1_w, out1_b, out2_w, out2_b, np_, nb, hdim):
    h2 = out1_w.shape[1]
    return pl.pallas_call(
        _readout_body,
        grid=(nb,),
        in_specs=[
            pl.BlockSpec((R, hdim), lambda b: (b, 0)),
            pl.BlockSpec((R, 1), lambda b: (b, 0)),
            pl.BlockSpec((hdim, h2), lambda b: (0, 0)),
            pl.BlockSpec((1, h2), lambda b: (0, 0)),
            pl.BlockSpec((h2, 1), lambda b: (0, 0)),
            pl.BlockSpec((1, 1), lambda b: (0, 0)),
        ],
        out_specs=pl.BlockSpec((1, GPAD), lambda b: (0, 0)),
        out_shape=jax.ShapeDtypeStruct((1, GPAD), jnp.float32),
    )(h, batch_pad_f, out1_w, out1_b, out2_w, out2_b)


def kernel(z, pos, batch, emb, i_mlp_w1, i_mlp_b1, i_mlp_w2, i_mlp_b2,
           i_lin1_w, i_lin2_w, i_lin2_b, i_lin_w, i_lin_b, out1_w, out1_b,
           out2_w, out2_b):
    n = pos.shape[0]
    hdim = emb.shape[1]
    nint = i_mlp_w1.shape[0]
    ngauss = i_mlp_w1.shape[1]
    ngraph = 100
    nb = -(-n // R)
    np_ = nb * R
    padn = np_ - n

    z = z.astype(jnp.int32)
    batch = batch.astype(jnp.int32)
    pos_pad = jnp.pad(pos.astype(jnp.float32), ((0, padn), (0, 0)))
    batch_pad = jnp.pad(batch, (0, padn), constant_values=GPAD - 1)
    z_pad = jnp.pad(z, (0, padn))
    batch_pad_f = batch_pad.astype(jnp.float32)[:, None]
    z_pad_f = z_pad.astype(jnp.float32)[:, None]

    row0 = jnp.arange(nb) * R
    g0 = batch_pad[row0]
    g1 = batch_pad[row0 + R - 1]
    cstart = jnp.searchsorted(batch_pad, g0, side="left")
    cend = jnp.searchsorted(batch_pad, g1, side="right")
    c0 = (cstart // CC) * CC
    nch = -(-(cend - c0) // CC)
    bounds = jnp.stack([c0, nch], axis=1).astype(jnp.int32)

    src, d, keep = _neighbors(pos_pad, batch_pad_f, bounds, np_, nb)
    src_flat = src.reshape(-1)
    d_flat = d.reshape(-1, 1)
    keep_flat = keep.reshape(-1, 1)

    emb_pad = jnp.pad(emb.astype(jnp.float32), ((0, ZPAD - emb.shape[0]),
                                                (0, 0)))
    offsets = jnp.linspace(0.0, CUTOFF, ngauss,
                           dtype=jnp.float32).reshape(1, ngauss)

    h, x1 = _embed(z_pad_f, emb_pad, i_lin1_w[0], np_, nb, hdim)
    for t in range(nint):
        g = _sc_gather(x1, src_flat, hdim)
        want_x1 = t + 1 < nint
        lin1n = i_lin1_w[t + 1] if want_x1 else i_lin1_w[0]
        h, x1 = _interaction(
            g, d_flat, keep_flat, h, offsets,
            i_mlp_w1[t], i_mlp_b1[t].reshape(1, -1),
            i_mlp_w2[t], i_mlp_b2[t].reshape(1, -1),
            i_lin2_w[t], i_lin2_b[t].reshape(1, -1),
            i_lin_w[t], i_lin_b[t].reshape(1, -1),
            lin1n, np_, nb, ngauss, hdim, want_x1)

    out = _readout(h, batch_pad_f, out1_w, out1_b.reshape(1, -1),
                   out2_w, out2_b.reshape(1, 1), np_, nb, hdim)
    return out[0, :ngraph].reshape(ngraph, 1)


# speedup vs baseline: 9.7377x; 1.8473x over previous
import functools
import math

import jax
import jax.numpy as jnp
from jax import lax
from jax.experimental import pallas as pl
from jax.experimental.pallas import tpu as pltpu
from jax.experimental.pallas import tpu_sc as plsc

R = 256
CC = 256
MAXNB = 32
CUTOFF = 10.0
GPAD = 128
ZPAD = 128


def _ssp(x):
    return jax.nn.softplus(x) - math.log(2.0)


def _neighbors_body(bounds_ref, pos_r_ref, pos_c_ref, batch_r_ref,
                    idx_ref, d_ref, keep_ref):
    b = pl.program_id(0)
    c0 = bounds_ref[b, 0]
    nch = bounds_ref[b, 1]

    pr = pos_r_ref[...]
    xr = pr[:, 0:1]
    yr = pr[:, 1:2]
    zr = pr[:, 2:3]
    sqr = xr * xr + yr * yr + zr * zr
    br = batch_r_ref[...]
    row_gid = b * R + lax.broadcasted_iota(jnp.int32, (R, 1), 0)

    big_pos = jnp.float32(1e9)
    lane32 = lax.broadcasted_iota(jnp.int32, (R, MAXNB), 1)
    posio = lax.broadcasted_iota(
        jnp.int32, (R, MAXNB + CC), 1).astype(jnp.float32)

    def chunk_body(j, carry):
        bd, bi = carry
        cs = pl.multiple_of(c0 + j * CC, CC)
        pc = pos_c_ref[:, pl.ds(cs, CC)]
        xc = pc[0:1, :]
        yc = pc[1:2, :]
        zc = pc[2:3, :]
        bc = pc[3:4, :]
        sqc = xc * xc + yc * yc + zc * zc
        d2 = sqr + sqc - 2.0 * (xr * xc + yr * yc + zr * zc)
        col_gid = cs + lax.broadcasted_iota(jnp.int32, (1, CC), 1)
        valid = ((br == bc) & (row_gid != col_gid)
                 & (d2 < CUTOFF * CUTOFF))
        cand_d2 = jnp.where(valid, d2, jnp.inf)
        cand_idx = jnp.broadcast_to(col_gid.astype(jnp.float32), (R, CC))

        v_d2 = jnp.concatenate([bd, cand_d2], axis=1)
        v_idx = jnp.concatenate([bi, cand_idx], axis=1)
        nb_d2 = jnp.full((R, MAXNB), jnp.inf, jnp.float32)
        nb_idx = jnp.zeros((R, MAXNB), jnp.float32)
        for i in range(MAXNB):
            m = jnp.min(v_d2, axis=1, keepdims=True)
            p = jnp.min(jnp.where(v_d2 == m, posio, big_pos),
                        axis=1, keepdims=True)
            chosen = posio == p
            ic = jnp.min(jnp.where(chosen, v_idx, big_pos),
                         axis=1, keepdims=True)
            nb_d2 = jnp.where(lane32 == i, m, nb_d2)
            nb_idx = jnp.where(lane32 == i, ic, nb_idx)
            v_d2 = jnp.where(chosen, jnp.inf, v_d2)
        return nb_d2, nb_idx

    init = (jnp.full((R, MAXNB), jnp.inf, jnp.float32),
            jnp.zeros((R, MAXNB), jnp.float32))
    bd, bi = lax.fori_loop(0, nch, chunk_body, init)

    keep = bd < jnp.inf
    idx_ref[...] = bi.astype(jnp.int32)
    d = jnp.sqrt(jnp.maximum(bd, 1e-12))
    cmask = 0.5 * (jnp.cos(d * (math.pi / CUTOFF)) + 1.0)
    keep_ref[...] = jnp.where(keep, cmask, 0.0)
    d_ref[...] = jnp.where(keep, d, 0.0)


def _neighbors(pos_pad, batch_pad_f, bounds, np_, nb):
    grid_spec = pltpu.PrefetchScalarGridSpec(
        num_scalar_prefetch=1,
        grid=(nb,),
        in_specs=[
            pl.BlockSpec((R, 3), lambda b, s: (b, 0)),
            pl.BlockSpec((8, np_), lambda b, s: (0, 0)),
            pl.BlockSpec((R, 1), lambda b, s: (b, 0)),
        ],
        out_specs=[
            pl.BlockSpec((R, MAXNB), lambda b, s: (b, 0)),
            pl.BlockSpec((R, MAXNB), lambda b, s: (b, 0)),
            pl.BlockSpec((R, MAXNB), lambda b, s: (b, 0)),
        ],
    )
    pos_cols = jnp.concatenate(
        [pos_pad.T, batch_pad_f.T, jnp.zeros((4, np_), jnp.float32)], axis=0)
    return pl.pallas_call(
        _neighbors_body,
        grid_spec=grid_spec,
        out_shape=[
            jax.ShapeDtypeStruct((np_, MAXNB), jnp.int32),
            jax.ShapeDtypeStruct((np_, MAXNB), jnp.float32),
            jax.ShapeDtypeStruct((np_, MAXNB), jnp.float32),
        ],
    )(bounds, pos_pad, pos_cols, batch_pad_f)


def _embed_body(z_ref, emb_ref, lin1_ref, h_ref, x1_ref):
    zc = z_ref[...]
    zio = lax.broadcasted_iota(jnp.int32, (1, ZPAD), 1).astype(jnp.float32)
    onehot = (zc == zio).astype(jnp.float32)
    h = jnp.dot(onehot, emb_ref[...],
                preferred_element_type=jnp.float32)
    h_ref[...] = h
    x1_ref[...] = jnp.dot(h, lin1_ref[...],
                          preferred_element_type=jnp.float32)


def _embed(z_pad_f, emb_pad, lin1_0, np_, nb, hdim):
    return pl.pallas_call(
        _embed_body,
        grid=(nb,),
        in_specs=[
            pl.BlockSpec((R, 1), lambda b: (b, 0)),
            pl.BlockSpec((ZPAD, hdim), lambda b: (0, 0)),
            pl.BlockSpec((hdim, hdim), lambda b: (0, 0)),
        ],
        out_specs=[
            pl.BlockSpec((R, hdim), lambda b: (b, 0)),
            pl.BlockSpec((R, hdim), lambda b: (b, 0)),
        ],
        out_shape=[
            jax.ShapeDtypeStruct((np_, hdim), jnp.float32),
            jax.ShapeDtypeStruct((np_, hdim), jnp.float32),
        ],
    )(z_pad_f, emb_pad, lin1_0)


def _sc_gather(table, idx, hdim):
    btot = idx.shape[0]
    info = plsc.get_sparse_core_info()
    nw = info.num_cores * info.num_subcores
    b_per_w = btot // nw
    ch = 512
    n_chunks = b_per_w // ch
    mesh = plsc.VectorSubcoreMesh(core_axis_name="c", subcore_axis_name="s")

    @functools.partial(
        pl.kernel,
        out_type=jax.ShapeDtypeStruct((btot, hdim), jnp.float32),
        mesh=mesh,
        scratch_types=[
            pltpu.VMEM((ch,), jnp.int32),
            pltpu.VMEM((ch, hdim), jnp.float32),
            pltpu.SemaphoreType.DMA,
        ],
        compiler_params=pltpu.CompilerParams(use_tc_tiling_on_sc=False),
    )
    def gather_kernel(table_hbm, idx_hbm, out_hbm, idx_v, rows_v, sem):
        wid = lax.axis_index("s") * info.num_cores + lax.axis_index("c")
        base = wid * b_per_w

        def body(c, carry):
            off = base + c * ch
            pltpu.sync_copy(idx_hbm.at[pl.ds(off, ch)], idx_v)
            pltpu.async_copy(table_hbm.at[idx_v], rows_v, sem).wait()
            pltpu.sync_copy(rows_v, out_hbm.at[pl.ds(off, ch)])
            return carry

        lax.fori_loop(0, n_chunks, body, 0)

    return gather_kernel(table, idx)


def _interaction_body(g_ref, d_ref, c_ref, h_ref, off_ref, w1_ref, b1_ref,
                      w2_ref, b2_ref, lin2w_ref, lin2b_ref, linw_ref,
                      linb_ref, lin1n_ref, h_out_ref, x1_out_ref, *,
                      ngauss, hdim, want_x1):
    dcol = d_ref[...]
    off = off_ref[...]
    step = CUTOFF / (ngauss - 1)
    coeff = -0.5 / (step * step)
    ea = jnp.exp(coeff * (dcol - off) ** 2)
    t1 = _ssp(jnp.dot(ea, w1_ref[...],
                      preferred_element_type=jnp.float32) + b1_ref[...])
    w = jnp.dot(t1, w2_ref[...],
                preferred_element_type=jnp.float32) + b2_ref[...]
    w = w * c_ref[...]
    msg = g_ref[...] * w
    aggr = jnp.sum(msg.reshape(R, MAXNB, hdim), axis=1)
    conv = jnp.dot(aggr, lin2w_ref[...],
                   preferred_element_type=jnp.float32) + lin2b_ref[...]
    hn = h_ref[...] + jnp.dot(_ssp(conv), linw_ref[...],
                              preferred_element_type=jnp.float32) + linb_ref[...]
    h_out_ref[...] = hn
    if want_x1:
        x1_out_ref[...] = jnp.dot(hn, lin1n_ref[...],
                                  preferred_element_type=jnp.float32)


def _interaction(g, d_flat, c_flat, h, offsets, w1, b1, w2, b2,
                 lin2w, lin2b, linw, linb, lin1n, np_, nb, ngauss, hdim,
                 want_x1):
    er = R * MAXNB
    out_shape = [jax.ShapeDtypeStruct((np_, hdim), jnp.float32)]
    out_specs = [pl.BlockSpec((R, hdim), lambda b: (b, 0))]
    if want_x1:
        out_shape.append(jax.ShapeDtypeStruct((np_, hdim), jnp.float32))
        out_specs.append(pl.BlockSpec((R, hdim), lambda b: (b, 0)))
    body = functools.partial(_interaction_body, ngauss=ngauss, hdim=hdim,
                             want_x1=want_x1)
    if not want_x1:
        def body2(g_ref, d_ref, c_ref, h_ref, off_ref, w1_ref, b1_ref,
                  w2_ref, b2_ref, lin2w_ref, lin2b_ref, linw_ref, linb_ref,
                  lin1n_ref, h_out_ref):
            body(g_ref, d_ref, c_ref, h_ref, off_ref, w1_ref, b1_ref,
                 w2_ref, b2_ref, lin2w_ref, lin2b_ref, linw_ref, linb_ref,
                 lin1n_ref, h_out_ref, None)
        kfn = body2
    else:
        kfn = body
    outs = pl.pallas_call(
        kfn,
        grid=(nb,),
        in_specs=[
            pl.BlockSpec((er, hdim), lambda b: (b, 0)),
            pl.BlockSpec((er, 1), lambda b: (b, 0)),
            pl.BlockSpec((er, 1), lambda b: (b, 0)),
            pl.BlockSpec((R, hdim), lambda b: (b, 0)),
            pl.BlockSpec((1, ngauss), lambda b: (0, 0)),
            pl.BlockSpec((ngauss, hdim), lambda b: (0, 0)),
            pl.BlockSpec((1, hdim), lambda b: (0, 0)),
            pl.BlockSpec((hdim, hdim), lambda b: (0, 0)),
            pl.BlockSpec((1, hdim), lambda b: (0, 0)),
            pl.BlockSpec((hdim, hdim), lambda b: (0, 0)),
            pl.BlockSpec((1, hdim), lambda b: (0, 0)),
            pl.BlockSpec((hdim, hdim), lambda b: (0, 0)),
            pl.BlockSpec((1, hdim), lambda b: (0, 0)),
            pl.BlockSpec((hdim, hdim), lambda b: (0, 0)),
        ],
        out_specs=out_specs,
        out_shape=out_shape,
    )(g, d_flat, c_flat, h, offsets, w1, b1, w2, b2, lin2w, lin2b,
      linw, linb, lin1n)
    if want_x1:
        return outs
    return outs[0], None


def _readout_body(h_ref, batch_ref, w1_ref, b1_ref, w2_ref, b2_ref, out_ref):
    @pl.when(pl.program_id(0) == 0)
    def _():
        out_ref[...] = jnp.zeros_like(out_ref)

    s = _ssp(jnp.dot(h_ref[...], w1_ref[...],
                     preferred_element_type=jnp.float32) + b1_ref[...])
    s = jnp.dot(s, w2_ref[...],
                preferred_element_type=jnp.float32) + b2_ref[...]
    gio = lax.broadcasted_iota(jnp.int32, (1, GPAD), 1).astype(jnp.float32)
    onehot = (batch_ref[...] == gio).astype(jnp.float32)
    out_ref[...] += jnp.sum(onehot * s, axis=0, keepdims=True)


def _readout(h, batch_pad_f, out1_w, out1_b, out2_w, out2_b, np_, nb, hdim):
    h2 = out1_w.shape[1]
    return pl.pallas_call(
        _readout_body,
        grid=(nb,),
        in_specs=[
            pl.BlockSpec((R, hdim), lambda b: (b, 0)),
            pl.BlockSpec((R, 1), lambda b: (b, 0)),
            pl.BlockSpec((hdim, h2), lambda b: (0, 0)),
            pl.BlockSpec((1, h2), lambda b: (0, 0)),
            pl.BlockSpec((h2, 1), lambda b: (0, 0)),
            pl.BlockSpec((1, 1), lambda b: (0, 0)),
        ],
        out_specs=pl.BlockSpec((1, GPAD), lambda b: (0, 0)),
        out_shape=jax.ShapeDtypeStruct((1, GPAD), jnp.float32),
    )(h, batch_pad_f, out1_w, out1_b, out2_w, out2_b)


def kernel(z, pos, batch, emb, i_mlp_w1, i_mlp_b1, i_mlp_w2, i_mlp_b2,
           i_lin1_w, i_lin2_w, i_lin2_b, i_lin_w, i_lin_b, out1_w, out1_b,
           out2_w, out2_b):
    n = pos.shape[0]
    hdim = emb.shape[1]
    nint = i_mlp_w1.shape[0]
    ngauss = i_mlp_w1.shape[1]
    ngraph = 100
    nb = -(-n // R)
    np_ = nb * R
    padn = np_ - n

    z = z.astype(jnp.int32)
    batch = batch.astype(jnp.int32)
    pos_pad = jnp.pad(pos.astype(jnp.float32), ((0, padn), (0, 0)))
    batch_pad = jnp.pad(batch, (0, padn), constant_values=GPAD - 1)
    z_pad = jnp.pad(z, (0, padn))
    batch_pad_f = batch_pad.astype(jnp.float32)[:, None]
    z_pad_f = z_pad.astype(jnp.float32)[:, None]

    row0 = jnp.arange(nb) * R
    g0 = batch_pad[row0]
    g1 = batch_pad[row0 + R - 1]
    cstart = jnp.searchsorted(batch_pad, g0, side="left")
    cend = jnp.searchsorted(batch_pad, g1, side="right")
    c0 = (cstart // CC) * CC
    nch = -(-(cend - c0) // CC)
    bounds = jnp.stack([c0, nch], axis=1).astype(jnp.int32)

    src, d, cmask = _neighbors(pos_pad, batch_pad_f, bounds, np_, nb)
    src_flat = src.reshape(-1)
    d_flat = d.reshape(-1, 1)
    c_flat = cmask.reshape(-1, 1)

    emb_pad = jnp.pad(emb.astype(jnp.float32), ((0, ZPAD - emb.shape[0]),
                                                (0, 0)))
    offsets = jnp.linspace(0.0, CUTOFF, ngauss,
                           dtype=jnp.float32).reshape(1, ngauss)

    h, x1 = _embed(z_pad_f, emb_pad, i_lin1_w[0], np_, nb, hdim)
    for t in range(nint):
        g = _sc_gather(x1, src_flat, hdim)
        want_x1 = t + 1 < nint
        lin1n = i_lin1_w[t + 1] if want_x1 else i_lin1_w[0]
        h, x1 = _interaction(
            g, d_flat, c_flat, h, offsets,
            i_mlp_w1[t], i_mlp_b1[t].reshape(1, -1),
            i_mlp_w2[t], i_mlp_b2[t].reshape(1, -1),
            i_lin2_w[t], i_lin2_b[t].reshape(1, -1),
            i_lin_w[t], i_lin_b[t].reshape(1, -1),
            lin1n, np_, nb, ngauss, hdim, want_x1)

    out = _readout(h, batch_pad_f, out1_w, out1_b.reshape(1, -1),
                   out2_w, out2_b.reshape(1, 1), np_, nb, hdim)
    return out[0, :ngraph].reshape(ngraph, 1)


# speedup vs baseline: 9.9862x; 1.0255x over previous
import functools
import math

import jax
import jax.numpy as jnp
from jax import lax
from jax.experimental import pallas as pl
from jax.experimental.pallas import tpu as pltpu
from jax.experimental.pallas import tpu_sc as plsc

R = 256
CC = 256
MAXNB = 32
CUTOFF = 10.0
GPAD = 128
ZPAD = 128


def _ssp(x):
    return jax.nn.softplus(x) - math.log(2.0)


def _neighbors_body(bounds_ref, pos_r_ref, pos_c_ref, batch_r_ref,
                    idx_ref, d_ref, keep_ref):
    b = pl.program_id(0)
    c0 = bounds_ref[b, 0]
    nch = bounds_ref[b, 1]

    pr = pos_r_ref[...]
    xr = pr[:, 0:1]
    yr = pr[:, 1:2]
    zr = pr[:, 2:3]
    sqr = xr * xr + yr * yr + zr * zr
    br = batch_r_ref[...]
    row_gid = b * R + lax.broadcasted_iota(jnp.int32, (R, 1), 0)

    big_pos = jnp.float32(1e9)
    lane32 = lax.broadcasted_iota(jnp.int32, (R, MAXNB), 1)
    posio = lax.broadcasted_iota(
        jnp.int32, (R, MAXNB + CC), 1).astype(jnp.float32)

    def chunk_body(j, carry):
        bd, bi = carry
        cs = pl.multiple_of(c0 + j * CC, CC)
        pc = pos_c_ref[:, pl.ds(cs, CC)]
        xc = pc[0:1, :]
        yc = pc[1:2, :]
        zc = pc[2:3, :]
        bc = pc[3:4, :]
        sqc = xc * xc + yc * yc + zc * zc
        d2 = sqr + sqc - 2.0 * (xr * xc + yr * yc + zr * zc)
        col_gid = cs + lax.broadcasted_iota(jnp.int32, (1, CC), 1)
        valid = ((br == bc) & (row_gid != col_gid)
                 & (d2 < CUTOFF * CUTOFF))
        cand_d2 = jnp.where(valid, d2, jnp.inf)
        cand_idx = jnp.broadcast_to(col_gid.astype(jnp.float32), (R, CC))

        v_d2 = jnp.concatenate([bd, cand_d2], axis=1)
        v_idx = jnp.concatenate([bi, cand_idx], axis=1)
        nb_d2 = jnp.full((R, MAXNB), jnp.inf, jnp.float32)
        nb_idx = jnp.zeros((R, MAXNB), jnp.float32)
        for i in range(MAXNB):
            m = jnp.min(v_d2, axis=1, keepdims=True)
            p = jnp.min(jnp.where(v_d2 == m, posio, big_pos),
                        axis=1, keepdims=True)
            chosen = posio == p
            ic = jnp.min(jnp.where(chosen, v_idx, big_pos),
                         axis=1, keepdims=True)
            nb_d2 = jnp.where(lane32 == i, m, nb_d2)
            nb_idx = jnp.where(lane32 == i, ic, nb_idx)
            v_d2 = jnp.where(chosen, jnp.inf, v_d2)
        return nb_d2, nb_idx

    init = (jnp.full((R, MAXNB), jnp.inf, jnp.float32),
            jnp.zeros((R, MAXNB), jnp.float32))
    bd, bi = lax.fori_loop(0, nch, chunk_body, init)

    keep = bd < jnp.inf
    idx_ref[...] = bi.astype(jnp.int32)
    d = jnp.sqrt(jnp.maximum(bd, 1e-12))
    cmask = 0.5 * (jnp.cos(d * (math.pi / CUTOFF)) + 1.0)
    keep_ref[...] = jnp.where(keep, cmask, 0.0)
    d_ref[...] = jnp.where(keep, d, 0.0)


def _neighbors(pos_pad, batch_pad_f, bounds, np_, nb):
    grid_spec = pltpu.PrefetchScalarGridSpec(
        num_scalar_prefetch=1,
        grid=(nb,),
        in_specs=[
            pl.BlockSpec((R, 3), lambda b, s: (b, 0)),
            pl.BlockSpec((8, np_), lambda b, s: (0, 0)),
            pl.BlockSpec((R, 1), lambda b, s: (b, 0)),
        ],
        out_specs=[
            pl.BlockSpec((R, MAXNB), lambda b, s: (b, 0)),
            pl.BlockSpec((R, MAXNB), lambda b, s: (b, 0)),
            pl.BlockSpec((R, MAXNB), lambda b, s: (b, 0)),
        ],
    )
    pos_cols = jnp.concatenate(
        [pos_pad.T, batch_pad_f.T, jnp.zeros((4, np_), jnp.float32)], axis=0)
    return pl.pallas_call(
        _neighbors_body,
        grid_spec=grid_spec,
        out_shape=[
            jax.ShapeDtypeStruct((np_, MAXNB), jnp.int32),
            jax.ShapeDtypeStruct((np_, MAXNB), jnp.float32),
            jax.ShapeDtypeStruct((np_, MAXNB), jnp.float32),
        ],
    )(bounds, pos_pad, pos_cols, batch_pad_f)


def _embed_body(z_ref, emb_ref, lin1_ref, h_ref, x1_ref):
    zc = z_ref[...]
    zio = lax.broadcasted_iota(jnp.int32, (1, ZPAD), 1).astype(jnp.float32)
    onehot = (zc == zio).astype(jnp.float32)
    h = jnp.dot(onehot, emb_ref[...],
                preferred_element_type=jnp.float32)
    h_ref[...] = h
    x1_ref[...] = jnp.dot(h, lin1_ref[...],
                          preferred_element_type=jnp.float32)


def _embed(z_pad_f, emb_pad, lin1_0, np_, nb, hdim):
    return pl.pallas_call(
        _embed_body,
        grid=(nb,),
        in_specs=[
            pl.BlockSpec((R, 1), lambda b: (b, 0)),
            pl.BlockSpec((ZPAD, hdim), lambda b: (0, 0)),
            pl.BlockSpec((hdim, hdim), lambda b: (0, 0)),
        ],
        out_specs=[
            pl.BlockSpec((R, hdim), lambda b: (b, 0)),
            pl.BlockSpec((R, hdim), lambda b: (b, 0)),
        ],
        out_shape=[
            jax.ShapeDtypeStruct((np_, hdim), jnp.float32),
            jax.ShapeDtypeStruct((np_, hdim), jnp.float32),
        ],
    )(z_pad_f, emb_pad, lin1_0)


def _sc_gather(table, idx, hdim):
    btot = idx.shape[0]
    info = plsc.get_sparse_core_info()
    nw = info.num_cores * info.num_subcores
    b_per_w = btot // nw
    ch = next(c for c in (640, 512, 320, 256, 160, 128, 80, 64, 40, 32, 16, 8)
              if b_per_w % c == 0)
    n_chunks = b_per_w // ch
    mesh = plsc.VectorSubcoreMesh(core_axis_name="c", subcore_axis_name="s")

    @functools.partial(
        pl.kernel,
        out_type=jax.ShapeDtypeStruct((btot, hdim), jnp.float32),
        mesh=mesh,
        scratch_types=[
            pltpu.VMEM((2, ch), jnp.int32),
            pltpu.VMEM((2, ch, hdim), jnp.float32),
            pltpu.SemaphoreType.DMA((2,)),
        ],
        compiler_params=pltpu.CompilerParams(use_tc_tiling_on_sc=False),
    )
    def gather_kernel(table_hbm, idx_hbm, out_hbm, idx_v, rows_v, sem):
        wid = lax.axis_index("s") * info.num_cores + lax.axis_index("c")
        base = wid * b_per_w

        pltpu.sync_copy(idx_hbm.at[pl.ds(base, ch)], idx_v.at[0])
        pltpu.async_copy(table_hbm.at[idx_v.at[0]], rows_v.at[0], sem.at[0])

        def body(c, carry):
            p = lax.rem(c, 2)
            q = 1 - p
            off = base + c * ch

            @pl.when(c < n_chunks)
            def _():
                pltpu.sync_copy(idx_hbm.at[pl.ds(off, ch)], idx_v.at[p])
                pltpu.async_copy(table_hbm.at[idx_v.at[p]], rows_v.at[p],
                                 sem.at[p])

            prev_off = base + (c - 1) * ch
            pltpu.make_async_copy(table_hbm.at[idx_v.at[q]], rows_v.at[q],
                                  sem.at[q]).wait()
            pltpu.sync_copy(rows_v.at[q], out_hbm.at[pl.ds(prev_off, ch)])
            return carry

        lax.fori_loop(1, n_chunks + 1, body, 0)

    return gather_kernel(table, idx)


def _interaction_body(g_ref, d_ref, c_ref, h_ref, off_ref, w1_ref, b1_ref,
                      w2_ref, b2_ref, lin2w_ref, lin2b_ref, linw_ref,
                      linb_ref, lin1n_ref, h_out_ref, x1_out_ref, *,
                      ngauss, hdim, want_x1):
    dcol = d_ref[...]
    off = off_ref[...]
    step = CUTOFF / (ngauss - 1)
    coeff = -0.5 / (step * step)
    ea = jnp.exp(coeff * (dcol - off) ** 2)
    t1 = _ssp(jnp.dot(ea, w1_ref[...],
                      preferred_element_type=jnp.float32) + b1_ref[...])
    w = jnp.dot(t1, w2_ref[...],
                preferred_element_type=jnp.float32) + b2_ref[...]
    w = w * c_ref[...]
    msg = g_ref[...] * w
    aggr = jnp.sum(msg.reshape(R, MAXNB, hdim), axis=1)
    conv = jnp.dot(aggr, lin2w_ref[...],
                   preferred_element_type=jnp.float32) + lin2b_ref[...]
    hn = h_ref[...] + jnp.dot(_ssp(conv), linw_ref[...],
                              preferred_element_type=jnp.float32) + linb_ref[...]
    h_out_ref[...] = hn
    if want_x1:
        x1_out_ref[...] = jnp.dot(hn, lin1n_ref[...],
                                  preferred_element_type=jnp.float32)


def _interaction(g, d_flat, c_flat, h, offsets, w1, b1, w2, b2,
                 lin2w, lin2b, linw, linb, lin1n, np_, nb, ngauss, hdim,
                 want_x1):
    er = R * MAXNB
    out_shape = [jax.ShapeDtypeStruct((np_, hdim), jnp.float32)]
    out_specs = [pl.BlockSpec((R, hdim), lambda b: (b, 0))]
    if want_x1:
        out_shape.append(jax.ShapeDtypeStruct((np_, hdim), jnp.float32))
        out_specs.append(pl.BlockSpec((R, hdim), lambda b: (b, 0)))
    body = functools.partial(_interaction_body, ngauss=ngauss, hdim=hdim,
                             want_x1=want_x1)
    if not want_x1:
        def body2(g_ref, d_ref, c_ref, h_ref, off_ref, w1_ref, b1_ref,
                  w2_ref, b2_ref, lin2w_ref, lin2b_ref, linw_ref, linb_ref,
                  lin1n_ref, h_out_ref):
            body(g_ref, d_ref, c_ref, h_ref, off_ref, w1_ref, b1_ref,
                 w2_ref, b2_ref, lin2w_ref, lin2b_ref, linw_ref, linb_ref,
                 lin1n_ref, h_out_ref, None)
        kfn = body2
    else:
        kfn = body
    outs = pl.pallas_call(
        kfn,
        grid=(nb,),
        in_specs=[
            pl.BlockSpec((er, hdim), lambda b: (b, 0)),
            pl.BlockSpec((er, 1), lambda b: (b, 0)),
            pl.BlockSpec((er, 1), lambda b: (b, 0)),
            pl.BlockSpec((R, hdim), lambda b: (b, 0)),
            pl.BlockSpec((1, ngauss), lambda b: (0, 0)),
            pl.BlockSpec((ngauss, hdim), lambda b: (0, 0)),
            pl.BlockSpec((1, hdim), lambda b: (0, 0)),
            pl.BlockSpec((hdim, hdim), lambda b: (0, 0)),
            pl.BlockSpec((1, hdim), lambda b: (0, 0)),
            pl.BlockSpec((hdim, hdim), lambda b: (0, 0)),
            pl.BlockSpec((1, hdim), lambda b: (0, 0)),
            pl.BlockSpec((hdim, hdim), lambda b: (0, 0)),
            pl.BlockSpec((1, hdim), lambda b: (0, 0)),
            pl.BlockSpec((hdim, hdim), lambda b: (0, 0)),
        ],
        out_specs=out_specs,
        out_shape=out_shape,
    )(g, d_flat, c_flat, h, offsets, w1, b1, w2, b2, lin2w, lin2b,
      linw, linb, lin1n)
    if want_x1:
        return outs
    return outs[0], None


def _readout_body(h_ref, batch_ref, w1_ref, b1_ref, w2_ref, b2_ref, out_ref):
    @pl.when(pl.program_id(0) == 0)
    def _():
        out_ref[...] = jnp.zeros_like(out_ref)

    s = _ssp(jnp.dot(h_ref[...], w1_ref[...],
                     preferred_element_type=jnp.float32) + b1_ref[...])
    s = jnp.dot(s, w2_ref[...],
                preferred_element_type=jnp.float32) + b2_ref[...]
    gio = lax.broadcasted_iota(jnp.int32, (1, GPAD), 1).astype(jnp.float32)
    onehot = (batch_ref[...] == gio).astype(jnp.float32)
    out_ref[...] += jnp.sum(onehot * s, axis=0, keepdims=True)


def _readout(h, batch_pad_f, out1_w, out1_b, out2_w, out2_b, np_, nb, hdim):
    h2 = out1_w.shape[1]
    return pl.pallas_call(
        _readout_body,
        grid=(nb,),
        in_specs=[
            pl.BlockSpec((R, hdim), lambda b: (b, 0)),
            pl.BlockSpec((R, 1), lambda b: (b, 0)),
            pl.BlockSpec((hdim, h2), lambda b: (0, 0)),
            pl.BlockSpec((1, h2), lambda b: (0, 0)),
            pl.BlockSpec((h2, 1), lambda b: (0, 0)),
            pl.BlockSpec((1, 1), lambda b: (0, 0)),
        ],
        out_specs=pl.BlockSpec((1, GPAD), lambda b: (0, 0)),
        out_shape=jax.ShapeDtypeStruct((1, GPAD), jnp.float32),
    )(h, batch_pad_f, out1_w, out1_b, out2_w, out2_b)


def kernel(z, pos, batch, emb, i_mlp_w1, i_mlp_b1, i_mlp_w2, i_mlp_b2,
           i_lin1_w, i_lin2_w, i_lin2_b, i_lin_w, i_lin_b, out1_w, out1_b,
           out2_w, out2_b):
    n = pos.shape[0]
    hdim = emb.shape[1]
    nint = i_mlp_w1.shape[0]
    ngauss = i_mlp_w1.shape[1]
    ngraph = 100
    nb = -(-n // R)
    np_ = nb * R
    padn = np_ - n

    z = z.astype(jnp.int32)
    batch = batch.astype(jnp.int32)
    pos_pad = jnp.pad(pos.astype(jnp.float32), ((0, padn), (0, 0)))
    batch_pad = jnp.pad(batch, (0, padn), constant_values=GPAD - 1)
    z_pad = jnp.pad(z, (0, padn))
    batch_pad_f = batch_pad.astype(jnp.float32)[:, None]
    z_pad_f = z_pad.astype(jnp.float32)[:, None]

    row0 = jnp.arange(nb) * R
    g0 = batch_pad[row0]
    g1 = batch_pad[row0 + R - 1]
    cstart = jnp.searchsorted(batch_pad, g0, side="left")
    cend = jnp.searchsorted(batch_pad, g1, side="right")
    c0 = (cstart // CC) * CC
    nch = -(-(cend - c0) // CC)
    bounds = jnp.stack([c0, nch], axis=1).astype(jnp.int32)

    src, d, cmask = _neighbors(pos_pad, batch_pad_f, bounds, np_, nb)
    src_flat = src.reshape(-1)
    d_flat = d.reshape(-1, 1)
    c_flat = cmask.reshape(-1, 1)

    emb_pad = jnp.pad(emb.astype(jnp.float32), ((0, ZPAD - emb.shape[0]),
                                                (0, 0)))
    offsets = jnp.linspace(0.0, CUTOFF, ngauss,
                           dtype=jnp.float32).reshape(1, ngauss)

    h, x1 = _embed(z_pad_f, emb_pad, i_lin1_w[0], np_, nb, hdim)
    for t in range(nint):
        g = _sc_gather(x1, src_flat, hdim)
        want_x1 = t + 1 < nint
        lin1n = i_lin1_w[t + 1] if want_x1 else i_lin1_w[0]
        h, x1 = _interaction(
            g, d_flat, c_flat, h, offsets,
            i_mlp_w1[t], i_mlp_b1[t].reshape(1, -1),
            i_mlp_w2[t], i_mlp_b2[t].reshape(1, -1),
            i_lin2_w[t], i_lin2_b[t].reshape(1, -1),
            i_lin_w[t], i_lin_b[t].reshape(1, -1),
            lin1n, np_, nb, ngauss, hdim, want_x1)

    out = _readout(h, batch_pad_f, out1_w, out1_b.reshape(1, -1),
                   out2_w, out2_b.reshape(1, 1), np_, nb, hdim)
    return out[0, :ngraph].reshape(ngraph, 1)


# speedup vs baseline: 10.3729x; 1.0387x over previous
import functools
import math

import jax
import jax.numpy as jnp
from jax import lax
from jax.experimental import pallas as pl
from jax.experimental.pallas import tpu as pltpu
from jax.experimental.pallas import tpu_sc as plsc

R = 256
CC = 256
MAXNB = 32
CUTOFF = 10.0
GPAD = 128
ZPAD = 128


def _ssp(x):
    return jax.nn.softplus(x) - math.log(2.0)


def _neighbors_body(bounds_ref, pos_r_ref, pos_c_ref, batch_r_ref,
                    idx_ref, d_ref, keep_ref):
    b = pl.program_id(0)
    c0 = bounds_ref[b, 0]
    nch = bounds_ref[b, 1]

    pr = pos_r_ref[...]
    xr = pr[:, 0:1]
    yr = pr[:, 1:2]
    zr = pr[:, 2:3]
    sqr = xr * xr + yr * yr + zr * zr
    br = batch_r_ref[...]
    row_gid = b * R + lax.broadcasted_iota(jnp.int32, (R, 1), 0)

    big_pos = jnp.float32(1e9)
    lane32 = lax.broadcasted_iota(jnp.int32, (R, MAXNB), 1)
    posio = lax.broadcasted_iota(
        jnp.int32, (R, MAXNB + CC), 1).astype(jnp.float32)

    def chunk_body(j, carry):
        bd, bi = carry
        cs = pl.multiple_of(c0 + j * CC, CC)
        pc = pos_c_ref[:, pl.ds(cs, CC)]
        xc = pc[0:1, :]
        yc = pc[1:2, :]
        zc = pc[2:3, :]
        bc = pc[3:4, :]
        sqc = xc * xc + yc * yc + zc * zc
        d2 = sqr + sqc - 2.0 * (xr * xc + yr * yc + zr * zc)
        col_gid = cs + lax.broadcasted_iota(jnp.int32, (1, CC), 1)
        valid = ((br == bc) & (row_gid != col_gid)
                 & (d2 < CUTOFF * CUTOFF))
        cand_d2 = jnp.where(valid, d2, jnp.inf)
        cand_idx = jnp.broadcast_to(col_gid.astype(jnp.float32), (R, CC))

        v_d2 = jnp.concatenate([bd, cand_d2], axis=1)
        v_idx = jnp.concatenate([bi, cand_idx], axis=1)
        nb_d2 = jnp.full((R, MAXNB), jnp.inf, jnp.float32)
        nb_idx = jnp.zeros((R, MAXNB), jnp.float32)
        for i in range(MAXNB):
            m = jnp.min(v_d2, axis=1, keepdims=True)
            p = jnp.min(jnp.where(v_d2 == m, posio, big_pos),
                        axis=1, keepdims=True)
            chosen = posio == p
            ic = jnp.min(jnp.where(chosen, v_idx, big_pos),
                         axis=1, keepdims=True)
            nb_d2 = jnp.where(lane32 == i, m, nb_d2)
            nb_idx = jnp.where(lane32 == i, ic, nb_idx)
            v_d2 = jnp.where(chosen, jnp.inf, v_d2)
        return nb_d2, nb_idx

    init = (jnp.full((R, MAXNB), jnp.inf, jnp.float32),
            jnp.zeros((R, MAXNB), jnp.float32))
    bd, bi = lax.fori_loop(0, nch, chunk_body, init)

    keep = bd < jnp.inf
    idx_ref[...] = bi.astype(jnp.int32)
    d = jnp.sqrt(jnp.maximum(bd, 1e-12))
    cmask = 0.5 * (jnp.cos(d * (math.pi / CUTOFF)) + 1.0)
    keep_ref[...] = jnp.where(keep, cmask, 0.0)
    d_ref[...] = jnp.where(keep, d, 0.0)


def _neighbors(pos_pad, batch_pad_f, bounds, np_, nb):
    grid_spec = pltpu.PrefetchScalarGridSpec(
        num_scalar_prefetch=1,
        grid=(nb,),
        in_specs=[
            pl.BlockSpec((R, 3), lambda b, s: (b, 0)),
            pl.BlockSpec((8, np_), lambda b, s: (0, 0)),
            pl.BlockSpec((R, 1), lambda b, s: (b, 0)),
        ],
        out_specs=[
            pl.BlockSpec((R, MAXNB), lambda b, s: (b, 0)),
            pl.BlockSpec((R, MAXNB), lambda b, s: (b, 0)),
            pl.BlockSpec((R, MAXNB), lambda b, s: (b, 0)),
        ],
    )
    pos_cols = jnp.concatenate(
        [pos_pad.T, batch_pad_f.T, jnp.zeros((4, np_), jnp.float32)], axis=0)
    return pl.pallas_call(
        _neighbors_body,
        grid_spec=grid_spec,
        out_shape=[
            jax.ShapeDtypeStruct((np_, MAXNB), jnp.int32),
            jax.ShapeDtypeStruct((np_, MAXNB), jnp.float32),
            jax.ShapeDtypeStruct((np_, MAXNB), jnp.float32),
        ],
    )(bounds, pos_pad, pos_cols, batch_pad_f)


def _embed_body(z_ref, emb_ref, lin1_ref, h_ref, x1_ref):
    zc = z_ref[...]
    zio = lax.broadcasted_iota(jnp.int32, (1, ZPAD), 1).astype(jnp.float32)
    onehot = (zc == zio).astype(jnp.float32)
    h = jnp.dot(onehot, emb_ref[...],
                preferred_element_type=jnp.float32)
    h_ref[...] = h
    x1_ref[...] = jnp.dot(h, lin1_ref[...],
                          preferred_element_type=jnp.float32)


def _embed(z_pad_f, emb_pad, lin1_0, np_, nb, hdim):
    return pl.pallas_call(
        _embed_body,
        grid=(nb,),
        in_specs=[
            pl.BlockSpec((R, 1), lambda b: (b, 0)),
            pl.BlockSpec((ZPAD, hdim), lambda b: (0, 0)),
            pl.BlockSpec((hdim, hdim), lambda b: (0, 0)),
        ],
        out_specs=[
            pl.BlockSpec((R, hdim), lambda b: (b, 0)),
            pl.BlockSpec((R, hdim), lambda b: (b, 0)),
        ],
        out_shape=[
            jax.ShapeDtypeStruct((np_, hdim), jnp.float32),
            jax.ShapeDtypeStruct((np_, hdim), jnp.float32),
        ],
    )(z_pad_f, emb_pad, lin1_0)


def _sc_gather(table, idx, hdim):
    btot = idx.shape[0]
    info = plsc.get_sparse_core_info()
    nw = info.num_cores * info.num_subcores
    b_per_w = btot // nw
    ch = next(c for c in (640, 512, 320, 256, 160, 128, 80, 64, 40, 32, 16, 8)
              if b_per_w % c == 0)
    n_chunks = b_per_w // ch
    mesh = plsc.VectorSubcoreMesh(core_axis_name="c", subcore_axis_name="s")

    @functools.partial(
        pl.kernel,
        out_type=jax.ShapeDtypeStruct((btot, hdim), jnp.float32),
        mesh=mesh,
        scratch_types=[
            pltpu.VMEM((2, ch), jnp.int32),
            pltpu.VMEM((2, ch, hdim), jnp.float32),
            pltpu.SemaphoreType.DMA((2,)),
        ],
        compiler_params=pltpu.CompilerParams(use_tc_tiling_on_sc=False),
    )
    def gather_kernel(table_hbm, idx_hbm, out_hbm, idx_v, rows_v, sem):
        wid = lax.axis_index("s") * info.num_cores + lax.axis_index("c")
        base = wid * b_per_w

        pltpu.sync_copy(idx_hbm.at[pl.ds(base, ch)], idx_v.at[0])
        pltpu.async_copy(table_hbm.at[idx_v.at[0]], rows_v.at[0], sem.at[0])

        def body(c, carry):
            p = lax.rem(c, 2)
            q = 1 - p
            off = base + c * ch

            @pl.when(c < n_chunks)
            def _():
                pltpu.sync_copy(idx_hbm.at[pl.ds(off, ch)], idx_v.at[p])
                pltpu.async_copy(table_hbm.at[idx_v.at[p]], rows_v.at[p],
                                 sem.at[p])

            prev_off = base + (c - 1) * ch
            pltpu.make_async_copy(table_hbm.at[idx_v.at[q]], rows_v.at[q],
                                  sem.at[q]).wait()
            pltpu.sync_copy(rows_v.at[q], out_hbm.at[pl.ds(prev_off, ch)])
            return carry

        lax.fori_loop(1, n_chunks + 1, body, 0)

    return gather_kernel(table, idx)


def _interaction_body(g_ref, d_ref, c_ref, h_ref, off_ref, w1_ref, b1_ref,
                      w2_ref, b2_ref, lin2w_ref, lin2b_ref, linw_ref,
                      linb_ref, lin1n_ref, h_out_ref, x1_out_ref, *,
                      ngauss, hdim, want_x1):
    er = R * MAXNB
    dcol = jnp.transpose(d_ref[...].reshape(1, er))
    off = off_ref[...]
    step = CUTOFF / (ngauss - 1)
    coeff = -0.5 / (step * step)
    ea = jnp.exp(coeff * (dcol - off) ** 2)
    t1 = _ssp(jnp.dot(ea, w1_ref[...],
                      preferred_element_type=jnp.float32) + b1_ref[...])
    w = jnp.dot(t1, w2_ref[...],
                preferred_element_type=jnp.float32) + b2_ref[...]
    w = w * jnp.transpose(c_ref[...].reshape(1, er))
    msg = g_ref[...] * w
    aggr = jnp.sum(msg.reshape(R, MAXNB, hdim), axis=1)
    conv = jnp.dot(aggr, lin2w_ref[...],
                   preferred_element_type=jnp.float32) + lin2b_ref[...]
    hn = h_ref[...] + jnp.dot(_ssp(conv), linw_ref[...],
                              preferred_element_type=jnp.float32) + linb_ref[...]
    h_out_ref[...] = hn
    if want_x1:
        x1_out_ref[...] = jnp.dot(hn, lin1n_ref[...],
                                  preferred_element_type=jnp.float32)


def _interaction(g, d_flat, c_flat, h, offsets, w1, b1, w2, b2,
                 lin2w, lin2b, linw, linb, lin1n, np_, nb, ngauss, hdim,
                 want_x1):
    er = R * MAXNB
    out_shape = [jax.ShapeDtypeStruct((np_, hdim), jnp.float32)]
    out_specs = [pl.BlockSpec((R, hdim), lambda b: (b, 0))]
    if want_x1:
        out_shape.append(jax.ShapeDtypeStruct((np_, hdim), jnp.float32))
        out_specs.append(pl.BlockSpec((R, hdim), lambda b: (b, 0)))
    body = functools.partial(_interaction_body, ngauss=ngauss, hdim=hdim,
                             want_x1=want_x1)
    if not want_x1:
        def body2(g_ref, d_ref, c_ref, h_ref, off_ref, w1_ref, b1_ref,
                  w2_ref, b2_ref, lin2w_ref, lin2b_ref, linw_ref, linb_ref,
                  lin1n_ref, h_out_ref):
            body(g_ref, d_ref, c_ref, h_ref, off_ref, w1_ref, b1_ref,
                 w2_ref, b2_ref, lin2w_ref, lin2b_ref, linw_ref, linb_ref,
                 lin1n_ref, h_out_ref, None)
        kfn = body2
    else:
        kfn = body
    outs = pl.pallas_call(
        kfn,
        grid=(nb,),
        in_specs=[
            pl.BlockSpec((er, hdim), lambda b: (b, 0)),
            pl.BlockSpec((1, 1, er), lambda b: (b, 0, 0)),
            pl.BlockSpec((1, 1, er), lambda b: (b, 0, 0)),
            pl.BlockSpec((R, hdim), lambda b: (b, 0)),
            pl.BlockSpec((1, ngauss), lambda b: (0, 0)),
            pl.BlockSpec((ngauss, hdim), lambda b: (0, 0)),
            pl.BlockSpec((1, hdim), lambda b: (0, 0)),
            pl.BlockSpec((hdim, hdim), lambda b: (0, 0)),
            pl.BlockSpec((1, hdim), lambda b: (0, 0)),
            pl.BlockSpec((hdim, hdim), lambda b: (0, 0)),
            pl.BlockSpec((1, hdim), lambda b: (0, 0)),
            pl.BlockSpec((hdim, hdim), lambda b: (0, 0)),
            pl.BlockSpec((1, hdim), lambda b: (0, 0)),
            pl.BlockSpec((hdim, hdim), lambda b: (0, 0)),
        ],
        out_specs=out_specs,
        out_shape=out_shape,
    )(g, d_flat, c_flat, h, offsets, w1, b1, w2, b2, lin2w, lin2b,
      linw, linb, lin1n)
    if want_x1:
        return outs
    return outs[0], None


def _readout_body(h_ref, batch_ref, w1_ref, b1_ref, w2_ref, b2_ref, out_ref):
    @pl.when(pl.program_id(0) == 0)
    def _():
        out_ref[...] = jnp.zeros_like(out_ref)

    s = _ssp(jnp.dot(h_ref[...], w1_ref[...],
                     preferred_element_type=jnp.float32) + b1_ref[...])
    s = jnp.dot(s, w2_ref[...],
                preferred_element_type=jnp.float32) + b2_ref[...]
    gio = lax.broadcasted_iota(jnp.int32, (1, GPAD), 1).astype(jnp.float32)
    onehot = (batch_ref[...] == gio).astype(jnp.float32)
    out_ref[...] += jnp.sum(onehot * s, axis=0, keepdims=True)


def _readout(h, batch_pad_f, out1_w, out1_b, out2_w, out2_b, np_, nb, hdim):
    h2 = out1_w.shape[1]
    return pl.pallas_call(
        _readout_body,
        grid=(nb,),
        in_specs=[
            pl.BlockSpec((R, hdim), lambda b: (b, 0)),
            pl.BlockSpec((R, 1), lambda b: (b, 0)),
            pl.BlockSpec((hdim, h2), lambda b: (0, 0)),
            pl.BlockSpec((1, h2), lambda b: (0, 0)),
            pl.BlockSpec((h2, 1), lambda b: (0, 0)),
            pl.BlockSpec((1, 1), lambda b: (0, 0)),
        ],
        out_specs=pl.BlockSpec((1, GPAD), lambda b: (0, 0)),
        out_shape=jax.ShapeDtypeStruct((1, GPAD), jnp.float32),
    )(h, batch_pad_f, out1_w, out1_b, out2_w, out2_b)


def kernel(z, pos, batch, emb, i_mlp_w1, i_mlp_b1, i_mlp_w2, i_mlp_b2,
           i_lin1_w, i_lin2_w, i_lin2_b, i_lin_w, i_lin_b, out1_w, out1_b,
           out2_w, out2_b):
    n = pos.shape[0]
    hdim = emb.shape[1]
    nint = i_mlp_w1.shape[0]
    ngauss = i_mlp_w1.shape[1]
    ngraph = 100
    nb = -(-n // R)
    np_ = nb * R
    padn = np_ - n

    z = z.astype(jnp.int32)
    batch = batch.astype(jnp.int32)
    pos_pad = jnp.pad(pos.astype(jnp.float32), ((0, padn), (0, 0)))
    batch_pad = jnp.pad(batch, (0, padn), constant_values=GPAD - 1)
    z_pad = jnp.pad(z, (0, padn))
    batch_pad_f = batch_pad.astype(jnp.float32)[:, None]
    z_pad_f = z_pad.astype(jnp.float32)[:, None]

    row0 = jnp.arange(nb) * R
    g0 = batch_pad[row0]
    g1 = batch_pad[row0 + R - 1]
    cstart = jnp.searchsorted(batch_pad, g0, side="left")
    cend = jnp.searchsorted(batch_pad, g1, side="right")
    c0 = (cstart // CC) * CC
    nch = -(-(cend - c0) // CC)
    bounds = jnp.stack([c0, nch], axis=1).astype(jnp.int32)

    src, d, cmask = _neighbors(pos_pad, batch_pad_f, bounds, np_, nb)
    src_flat = src.reshape(-1)
    d_flat = d.reshape(nb, 1, R * MAXNB)
    c_flat = cmask.reshape(nb, 1, R * MAXNB)

    emb_pad = jnp.pad(emb.astype(jnp.float32), ((0, ZPAD - emb.shape[0]),
                                                (0, 0)))
    offsets = jnp.linspace(0.0, CUTOFF, ngauss,
                           dtype=jnp.float32).reshape(1, ngauss)

    h, x1 = _embed(z_pad_f, emb_pad, i_lin1_w[0], np_, nb, hdim)
    for t in range(nint):
        g = _sc_gather(x1, src_flat, hdim)
        want_x1 = t + 1 < nint
        lin1n = i_lin1_w[t + 1] if want_x1 else i_lin1_w[0]
        h, x1 = _interaction(
            g, d_flat, c_flat, h, offsets,
            i_mlp_w1[t], i_mlp_b1[t].reshape(1, -1),
            i_mlp_w2[t], i_mlp_b2[t].reshape(1, -1),
            i_lin2_w[t], i_lin2_b[t].reshape(1, -1),
            i_lin_w[t], i_lin_b[t].reshape(1, -1),
            lin1n, np_, nb, ngauss, hdim, want_x1)

    out = _readout(h, batch_pad_f, out1_w, out1_b.reshape(1, -1),
                   out2_w, out2_b.reshape(1, 1), np_, nb, hdim)
    return out[0, :ngraph].reshape(ngraph, 1)
